# baseline (speedup 1.0000x reference)
import functools

import jax
import jax.numpy as jnp
from jax import lax
from jax.experimental import pallas as pl
from jax.experimental.pallas import tpu as pltpu
from jax.experimental.pallas import tpu_sc as plsc

N_SPARSE = 26
N_CONT = 13
N_COLS = N_SPARSE + N_CONT
VOCAB = 100000
EMBED_DIM = 32
BATCH = 16384
OUT_W = N_SPARSE * EMBED_DIM + N_CONT
WIDE = 128
VOCAB_W = VOCAB // 4

NC = 2
NS = 16
NW = NC * NS
ROWS_PER_W = BATCH // NW
NB = 32
STAGE = 128
N_STAGE = ROWS_PER_W // STAGE


def _make_kernel():
    mesh = plsc.VectorSubcoreMesh(core_axis_name="c", subcore_axis_name="s")

    @functools.partial(
        pl.kernel,
        mesh=mesh,
        out_type=jax.ShapeDtypeStruct((BATCH, OUT_W), jnp.float32),
        compiler_params=pltpu.CompilerParams(needs_layout_passes=False),
        scratch_types=[
            pltpu.VMEM((N_COLS, STAGE), jnp.int32),
            pltpu.VMEM((N_SPARSE, NB), jnp.int32),
            pltpu.VMEM((NB, WIDE), jnp.float32),
            pltpu.VMEM((NB, WIDE), jnp.float32),
            pltpu.VMEM((NB, OUT_W), jnp.float32),
            pltpu.SemaphoreType.DMA,
            pltpu.SemaphoreType.DMA,
        ],
    )
    def k(xt_hbm, w_hbm, out_hbm, x_v, idx_v, gA, gB, row_v, semA, semB):
        wid = lax.axis_index("s") * NC + lax.axis_index("c")
        lane = lax.iota(jnp.int32, 16)

        def fire(i, gbuf, sem):
            return pltpu.async_copy(w_hbm.at[idx_v.at[i]], gbuf, sem)

        def drain(i, gbuf, sem):
            pltpu.make_async_copy(w_hbm.at[idx_v.at[i]], gbuf, sem).wait()

        def extract(i, gbuf, h0):
            for q in range(NB // 16):
                xq = plsc.load_gather(
                    x_v, [jnp.full((16,), 0, jnp.int32) + i, h0 + q * 16 + lane]
                )
                off = (xq & 3) * EMBED_DIM
                p_vec = q * 16 + lane
                ci = i * EMBED_DIM

                def body(j, carry):
                    val = plsc.load_gather(gbuf, [p_vec, off + j])
                    plsc.store_scatter(
                        row_v, [p_vec, jnp.full((16,), 0, jnp.int32) + ci + j], val
                    )
                    return carry

                lax.fori_loop(0, EMBED_DIM, body, 0)

        def stage_group(st, carry):
            c0 = pl.multiple_of(wid * ROWS_PER_W + st * STAGE, STAGE)
            pltpu.sync_copy(xt_hbm.at[:, pl.ds(c0, STAGE)], x_v)
            for blk in range(STAGE // NB):
                h0 = blk * NB
                b0 = c0 + h0

                def idx_body(i, carry):
                    for q in range(NB // 16):
                        xq = plsc.load_gather(
                            x_v,
                            [jnp.full((16,), 0, jnp.int32) + i, h0 + q * 16 + lane],
                        )
                        idx_v[i, pl.ds(q * 16, 16)] = i * VOCAB_W + (xq >> 2)
                    return carry

                lax.fori_loop(0, N_SPARSE, idx_body, 0)

                for j in range(N_CONT):
                    for q in range(NB // 16):
                        xc = x_v[N_SPARSE + j, pl.ds(h0 + q * 16, 16)]
                        plsc.store_scatter(
                            row_v,
                            [q * 16 + lane, jnp.full((16,), 832 + j, jnp.int32)],
                            xc.astype(jnp.float32),
                        )

                fire(0, gA, semA)

                def pipe(t, carry):
                    ia = 2 * t
                    fire(ia + 1, gB, semB)
                    drain(ia, gA, semA)
                    extract(ia, gA, h0)
                    fire(ia + 2, gA, semA)
                    drain(ia + 1, gB, semB)
                    extract(ia + 1, gB, h0)
                    return carry

                lax.fori_loop(0, N_SPARSE // 2 - 1, pipe, 0)
                fire(N_SPARSE - 1, gB, semB)
                drain(N_SPARSE - 2, gA, semA)
                extract(N_SPARSE - 2, gA, h0)
                drain(N_SPARSE - 1, gB, semB)
                extract(N_SPARSE - 1, gB, h0)

                pltpu.sync_copy(row_v, out_hbm.at[pl.ds(b0, NB), :])
            return carry

        lax.fori_loop(0, N_STAGE, stage_group, 0)

    return k


_kernel_fn = _make_kernel()


def kernel(X, W):
    xt = X.T
    w_wide = W.reshape(N_SPARSE * VOCAB // 4, WIDE)
    return _kernel_fn(xt, w_wide)


# speedup vs baseline: 1.5995x; 1.5995x over previous
import functools

import jax
import jax.numpy as jnp
from jax import lax
from jax.experimental import pallas as pl
from jax.experimental.pallas import tpu as pltpu
from jax.experimental.pallas import tpu_sc as plsc

N_SPARSE = 26
N_CONT = 13
N_COLS = N_SPARSE + N_CONT
VOCAB = 100000
EMBED_DIM = 32
BATCH = 16384
OUT_W = N_SPARSE * EMBED_DIM + N_CONT
WIDE = 128
VBLK = 4096
N_VBLK = -(-VOCAB // VBLK)
TROWS = N_VBLK * (VBLK // 4)

NC = 2
NS = 16
NW = NC * NS
ROWS_PER_W = BATCH // NW
NB = 32
STAGE = 128
N_STAGE = ROWS_PER_W // STAGE


def _make_kernel():
    mesh = plsc.VectorSubcoreMesh(core_axis_name="c", subcore_axis_name="s")

    @functools.partial(
        pl.kernel,
        mesh=mesh,
        out_type=jax.ShapeDtypeStruct((OUT_W, BATCH), jnp.float32),
        compiler_params=pltpu.CompilerParams(needs_layout_passes=False),
        scratch_types=[
            pltpu.VMEM((N_COLS, STAGE), jnp.int32),
            pltpu.VMEM((N_SPARSE, NB), jnp.int32),
            pltpu.VMEM((NB, WIDE), jnp.float32),
            pltpu.VMEM((NB, WIDE), jnp.float32),
            pltpu.VMEM((OUT_W, STAGE), jnp.float32),
            pltpu.SemaphoreType.DMA,
            pltpu.SemaphoreType.DMA,
        ],
    )
    def k(xt_hbm, w_hbm, out_hbm, x_v, idx_v, gA, gB, row_v, semA, semB):
        wid = lax.axis_index("s") * NC + lax.axis_index("c")
        lane = lax.iota(jnp.int32, 16)
        zero16 = jnp.zeros((16,), jnp.int32)

        def fire(i, gbuf, sem):
            pltpu.async_copy(w_hbm.at[idx_v.at[i]], gbuf, sem)

        def drain(i, gbuf, sem):
            pltpu.make_async_copy(w_hbm.at[idx_v.at[i]], gbuf, sem).wait()

        def extract(i, gbuf, h0):
            civ = zero16 + i * EMBED_DIM
            for q in range(NB // 16):
                xq = plsc.load_gather(x_v, [zero16 + i, h0 + q * 16 + lane])
                off = ((xq >> 10) & 3) * EMBED_DIM
                p_vec = q * 16 + lane
                colv = h0 + q * 16 + lane
                for j0 in range(0, EMBED_DIM, 4):
                    vals = [
                        plsc.load_gather(gbuf, [p_vec, off + (j0 + d)])
                        for d in range(4)
                    ]
                    for d in range(4):
                        plsc.store_scatter(row_v, [civ + (j0 + d), colv], vals[d])

        def stage_group(st, carry):
            c0 = pl.multiple_of(wid * ROWS_PER_W + st * STAGE, STAGE)
            pltpu.sync_copy(xt_hbm.at[:, pl.ds(c0, STAGE)], x_v)

            for jc in range(N_CONT):
                for q in range(STAGE // 16):
                    colv = q * 16 + lane
                    xc = x_v[N_SPARSE + jc, pl.ds(q * 16, 16)]
                    plsc.store_scatter(
                        row_v, [zero16 + (832 + jc), colv],
                        xc.astype(jnp.float32),
                    )

            for blk in range(STAGE // NB):
                h0 = blk * NB

                def idx_body(i, carry):
                    for q in range(NB // 16):
                        xq = plsc.load_gather(
                            x_v, [zero16 + i, h0 + q * 16 + lane]
                        )
                        idx_v[i, pl.ds(q * 16, 16)] = (
                            i * TROWS + ((xq >> 12) << 10) + (xq & 1023)
                        )
                    return carry

                lax.fori_loop(0, N_SPARSE, idx_body, 0)

                fire(0, gA, semA)

                def pipe(t, carry):
                    ia = 2 * t
                    fire(ia + 1, gB, semB)
                    drain(ia, gA, semA)
                    extract(ia, gA, h0)
                    fire(ia + 2, gA, semA)
                    drain(ia + 1, gB, semB)
                    extract(ia + 1, gB, h0)
                    return carry

                lax.fori_loop(0, N_SPARSE // 2 - 1, pipe, 0)
                fire(N_SPARSE - 1, gB, semB)
                drain(N_SPARSE - 2, gA, semA)
                extract(N_SPARSE - 2, gA, h0)
                drain(N_SPARSE - 1, gB, semB)
                extract(N_SPARSE - 1, gB, h0)

            pltpu.sync_copy(row_v, out_hbm.at[:, pl.ds(c0, STAGE)])
            return carry

        lax.fori_loop(0, N_STAGE, stage_group, 0)

    return k


_kernel_fn = _make_kernel()


def _tr_body(in_ref, out_ref):
    xt = in_ref[0].T
    parts = [
        xt[k * (VBLK // 4):(k + 1) * (VBLK // 4)] for k in range(4)
    ]
    out_ref[...] = jnp.concatenate(parts, axis=1)


def _tc_transpose_regroup(wt3):
    return pl.pallas_call(
        _tr_body,
        grid=(N_SPARSE, N_VBLK),
        in_specs=[pl.BlockSpec((1, EMBED_DIM, VBLK), lambda i, j: (i, 0, j))],
        out_specs=pl.BlockSpec((VBLK // 4, WIDE),
                               lambda i, j: (i * N_VBLK + j, 0)),
        out_shape=jax.ShapeDtypeStruct((N_SPARSE * TROWS, WIDE), jnp.float32),
    )(wt3)


def kernel(X, W):
    xt = X.T
    wt3 = W.transpose(0, 2, 1)
    w_wide = _tc_transpose_regroup(wt3)
    out_t = _kernel_fn(xt, w_wide)
    return out_t.T


# speedup vs baseline: 1.6108x; 1.0070x over previous
import functools

import jax
import jax.numpy as jnp
from jax import lax
from jax.experimental import pallas as pl
from jax.experimental.pallas import tpu as pltpu
from jax.experimental.pallas import tpu_sc as plsc

N_SPARSE = 26
N_CONT = 13
N_COLS = N_SPARSE + N_CONT
VOCAB = 100000
EMBED_DIM = 32
BATCH = 16384
OUT_W = N_SPARSE * EMBED_DIM + N_CONT
WIDE = 128
VBLK = 4096
N_VBLK = -(-VOCAB // VBLK)
TROWS = N_VBLK * (VBLK // 4)

NC = 2
NS = 16
NW = NC * NS
ROWS_PER_W = BATCH // NW
NB = 32
STAGE = 128
N_STAGE = ROWS_PER_W // STAGE


def _make_kernel():
    mesh = plsc.VectorSubcoreMesh(core_axis_name="c", subcore_axis_name="s")

    @functools.partial(
        pl.kernel,
        mesh=mesh,
        out_type=jax.ShapeDtypeStruct((OUT_W, BATCH), jnp.float32),
        compiler_params=pltpu.CompilerParams(needs_layout_passes=False),
        scratch_types=[
            pltpu.VMEM((N_COLS, STAGE), jnp.int32),
            pltpu.VMEM((N_SPARSE, NB), jnp.int32),
            pltpu.VMEM((NB, WIDE), jnp.float32),
            pltpu.VMEM((NB, WIDE), jnp.float32),
            pltpu.VMEM((OUT_W, STAGE), jnp.float32),
            pltpu.SemaphoreType.DMA,
            pltpu.SemaphoreType.DMA,
        ],
    )
    def k(xt_hbm, w_hbm, out_hbm, x_v, idx_v, gA, gB, row_v, semA, semB):
        wid = lax.axis_index("s") * NC + lax.axis_index("c")
        lane = lax.iota(jnp.int32, 16)
        zero16 = jnp.zeros((16,), jnp.int32)

        def fire(i, gbuf, sem):
            pltpu.async_copy(w_hbm.at[idx_v.at[i]], gbuf, sem)

        def drain(i, gbuf, sem):
            pltpu.make_async_copy(w_hbm.at[idx_v.at[i]], gbuf, sem).wait()

        def extract(i, gbuf, h0):
            civ = zero16 + i * EMBED_DIM
            for q in range(NB // 16):
                xq = plsc.load_gather(x_v, [zero16 + i, h0 + q * 16 + lane])
                off = ((xq >> 10) & 3) * EMBED_DIM
                p_vec = q * 16 + lane
                colv = h0 + q * 16 + lane
                for j0 in range(0, EMBED_DIM, 8):
                    vals = [
                        plsc.load_gather(gbuf, [p_vec, off + (j0 + d)])
                        for d in range(8)
                    ]
                    for d in range(8):
                        plsc.store_scatter(row_v, [civ + (j0 + d), colv], vals[d])

        def stage_group(st, carry):
            c0 = pl.multiple_of(wid * ROWS_PER_W + st * STAGE, STAGE)
            pltpu.sync_copy(xt_hbm.at[:, pl.ds(c0, STAGE)], x_v)

            for jc in range(N_CONT):
                for q in range(STAGE // 16):
                    colv = q * 16 + lane
                    xc = x_v[N_SPARSE + jc, pl.ds(q * 16, 16)]
                    plsc.store_scatter(
                        row_v, [zero16 + (832 + jc), colv],
                        xc.astype(jnp.float32),
                    )

            for blk in range(STAGE // NB):
                h0 = blk * NB

                def idx_body(i, carry):
                    for q in range(NB // 16):
                        xq = plsc.load_gather(
                            x_v, [zero16 + i, h0 + q * 16 + lane]
                        )
                        idx_v[i, pl.ds(q * 16, 16)] = (
                            i * TROWS + ((xq >> 12) << 10) + (xq & 1023)
                        )
                    return carry

                lax.fori_loop(0, N_SPARSE, idx_body, 0)

                fire(0, gA, semA)

                def pipe(t, carry):
                    ia = 2 * t
                    fire(ia + 1, gB, semB)
                    drain(ia, gA, semA)
                    extract(ia, gA, h0)
                    fire(ia + 2, gA, semA)
                    drain(ia + 1, gB, semB)
                    extract(ia + 1, gB, h0)
                    return carry

                lax.fori_loop(0, N_SPARSE // 2 - 1, pipe, 0)
                fire(N_SPARSE - 1, gB, semB)
                drain(N_SPARSE - 2, gA, semA)
                extract(N_SPARSE - 2, gA, h0)
                drain(N_SPARSE - 1, gB, semB)
                extract(N_SPARSE - 1, gB, h0)

            pltpu.sync_copy(row_v, out_hbm.at[:, pl.ds(c0, STAGE)])
            return carry

        lax.fori_loop(0, N_STAGE, stage_group, 0)

    return k


_kernel_fn = _make_kernel()


def _tr_body(in_ref, out_ref):
    xt = in_ref[0].T
    parts = [
        xt[k * (VBLK // 4):(k + 1) * (VBLK // 4)] for k in range(4)
    ]
    out_ref[...] = jnp.concatenate(parts, axis=1)


def _tc_transpose_regroup(wt3):
    return pl.pallas_call(
        _tr_body,
        grid=(N_SPARSE, N_VBLK),
        in_specs=[pl.BlockSpec((1, EMBED_DIM, VBLK), lambda i, j: (i, 0, j))],
        out_specs=pl.BlockSpec((VBLK // 4, WIDE),
                               lambda i, j: (i * N_VBLK + j, 0)),
        out_shape=jax.ShapeDtypeStruct((N_SPARSE * TROWS, WIDE), jnp.float32),
    )(wt3)


def kernel(X, W):
    xt = X.T
    wt3 = W.transpose(0, 2, 1)
    w_wide = _tc_transpose_regroup(wt3)
    out_t = _kernel_fn(xt, w_wide)
    return out_t.T
